# baseline (speedup 1.0000x reference)
import functools

import jax
import jax.numpy as jnp
from jax import lax
from jax.experimental import pallas as pl
from jax.experimental.pallas import tpu as pltpu
from jax.experimental.pallas import tpu_sc as plsc

VOCAB = 50257
EMBED = 768
B_TOTAL = 4 * 2048
NUM_WORKERS = 32
B_PER_W = B_TOTAL // NUM_WORKERS
CHUNK = 64
N_CHUNKS = B_PER_W // CHUNK


@functools.partial(
    pl.kernel,
    mesh=plsc.VectorSubcoreMesh(core_axis_name="c", subcore_axis_name="s"),
    out_type=jax.ShapeDtypeStruct((B_TOTAL, EMBED), jnp.float32),
    scratch_types=[
        pltpu.VMEM((N_CHUNKS, CHUNK), jnp.int32),
        pltpu.VMEM((CHUNK, EMBED), jnp.float32),
        pltpu.VMEM((CHUNK, EMBED), jnp.float32),
        pltpu.SemaphoreType.DMA,
        pltpu.SemaphoreType.DMA,
    ],
)
def _embed_lookup(table_hbm, idx_hbm, out_hbm, idx_v, rows_a, rows_b, gsem_a, gsem_b):
    c = lax.axis_index("c")
    s = lax.axis_index("s")
    wid = s * 2 + c
    base = wid * B_PER_W
    bufs = (rows_a, rows_b)
    gsems = (gsem_a, gsem_b)
    pltpu.sync_copy(idx_hbm.at[wid], idx_v)
    gathers = [None] * N_CHUNKS
    gathers[0] = pltpu.async_copy(table_hbm.at[idx_v.at[0]], bufs[0], gsems[0])
    for j in range(N_CHUNKS):
        b = j & 1
        gathers[j].wait()
        if j + 1 < N_CHUNKS:
            gathers[j + 1] = pltpu.async_copy(
                table_hbm.at[idx_v.at[j + 1]], bufs[b ^ 1], gsems[b ^ 1])
        pltpu.sync_copy(bufs[b], out_hbm.at[pl.ds(base + j * CHUNK, CHUNK)])


def kernel(x, table):
    idx = x.reshape(NUM_WORKERS, N_CHUNKS, CHUNK).astype(jnp.int32)
    out = _embed_lookup(table, idx)
    return out.reshape(x.shape[0], x.shape[1], EMBED)


# speedup vs baseline: 1.0576x; 1.0576x over previous
import functools

import jax
import jax.numpy as jnp
from jax import lax
from jax.experimental import pallas as pl
from jax.experimental.pallas import tpu as pltpu
from jax.experimental.pallas import tpu_sc as plsc

VOCAB = 50257
EMBED = 768
B_TOTAL = 4 * 2048
NUM_WORKERS = 32
B_PER_W = B_TOTAL // NUM_WORKERS
CHUNK = 32
N_CHUNKS = B_PER_W // CHUNK
NBUF = 4
PRIME = NBUF - 1


@functools.partial(
    pl.kernel,
    mesh=plsc.VectorSubcoreMesh(core_axis_name="c", subcore_axis_name="s"),
    out_type=jax.ShapeDtypeStruct((B_TOTAL, EMBED), jnp.float32),
    scratch_types=[
        pltpu.VMEM((N_CHUNKS, CHUNK), jnp.int32),
        pltpu.VMEM((NBUF, CHUNK, EMBED), jnp.float32),
    ] + [pltpu.SemaphoreType.DMA] * (2 * NBUF),
)
def _embed_lookup(table_hbm, idx_hbm, out_hbm, idx_v, rows_v, *sems):
    gsems, wsems = sems[:NBUF], sems[NBUF:]
    c = lax.axis_index("c")
    s = lax.axis_index("s")
    wid = s * 2 + c
    base = wid * B_PER_W
    pltpu.sync_copy(idx_hbm.at[wid], idx_v)
    gathers = [None] * N_CHUNKS
    writes = [None] * N_CHUNKS
    for j in range(PRIME):
        gathers[j] = pltpu.async_copy(
            table_hbm.at[idx_v.at[j]], rows_v.at[j % NBUF], gsems[j % NBUF])
    for j in range(N_CHUNKS):
        b = j % NBUF
        gathers[j].wait()
        writes[j] = pltpu.async_copy(
            rows_v.at[b], out_hbm.at[pl.ds(base + j * CHUNK, CHUNK)], wsems[b])
        nj = j + PRIME
        if nj < N_CHUNKS:
            if j >= 1:
                writes[j - 1].wait()
            gathers[nj] = pltpu.async_copy(
                table_hbm.at[idx_v.at[nj]], rows_v.at[nj % NBUF], gsems[nj % NBUF])
    for j in range(N_CHUNKS - PRIME - 1, N_CHUNKS):
        if writes[j] is not None and j >= N_CHUNKS - PRIME - 1:
            writes[j].wait()


def kernel(x, table):
    idx = x.reshape(NUM_WORKERS, N_CHUNKS, CHUNK).astype(jnp.int32)
    out = _embed_lookup(table, idx)
    return out.reshape(x.shape[0], x.shape[1], EMBED)
